# baseline (speedup 1.0000x reference)
import functools

import jax
import jax.numpy as jnp
from jax import lax
from jax.experimental import pallas as pl
from jax.experimental.pallas import tpu as pltpu
from jax.experimental.pallas import tpu_sc as plsc

N = 10000
E = 320000
D_IN = 128
H = 64
G = 128

NC = 2
NS = 16
NW = NC * NS
WIN = 160
NWIN = 64
EPAD = NW * NWIN * WIN - E
ROWS_ACC = 10112
ROWS_D = 10240

ACC_PER_SUB = ROWS_ACC // NS
D_PER_SUB = ROWS_D // NS
STRIPE = 640
TAIL = N - STRIPE * (NS - 1)

_MESH = dict(core_axis_name="c", subcore_axis_name="s",
             num_cores=NC, num_subcores=NS)
_SC_PARAMS = pltpu.CompilerParams(use_tc_tiling_on_sc=False)


def _deg_body(dstp, zeros1, out, dacc, didx, ones):
    c = lax.axis_index("c")
    s = lax.axis_index("s")
    wid = s * NC + c
    pltpu.sync_copy(zeros1.at[pl.ds(s * D_PER_SUB, D_PER_SUB)],
                    dacc.at[pl.ds(s * D_PER_SUB, D_PER_SUB)])
    pltpu.sync_copy(dstp.at[wid], didx)
    for i in range(WIN // 16):
        ones[pl.ds(i * 16, 16)] = jnp.ones((16,), jnp.float32)
    plsc.subcore_barrier()

    def win(w, carry):
        pltpu.sync_copy(ones, dacc.at[didx.at[w]], add=True)
        return carry

    lax.fori_loop(0, NWIN, win, 0, unroll=False)
    plsc.subcore_barrier()
    pltpu.sync_copy(dacc.at[pl.ds(s * D_PER_SUB, D_PER_SUB)],
                    out.at[pl.ds(c * ROWS_D + s * D_PER_SUB, D_PER_SUB)])


def _deg_kernel(dstp, zeros1):
    return pl.kernel(
        _deg_body,
        out_type=jax.ShapeDtypeStruct((NC * ROWS_D,), jnp.float32),
        mesh=plsc.VectorSubcoreMesh(**_MESH),
        compiler_params=_SC_PARAMS,
        scratch_types=[
            pltpu.VMEM_SHARED((ROWS_D,), jnp.float32),
            pltpu.VMEM((NWIN, WIN), jnp.int32),
            pltpu.VMEM((WIN,), jnp.float32),
        ],
    )(dstp, zeros1)


def _agg_body(g_hbm, zeros2, srcp, dstp, out, g_tbl, acc, sidx, didx,
              stg_a, stg_b, sem_a, sem_b):
    c = lax.axis_index("c")
    s = lax.axis_index("s")
    wid = s * NC + c
    @pl.when(s < NS - 1)
    def _():
        pltpu.sync_copy(g_hbm.at[pl.ds(s * STRIPE, STRIPE)],
                        g_tbl.at[pl.ds(s * STRIPE, STRIPE)])
        @pl.when(c == 0)
        def _():
            pltpu.sync_copy(g_hbm.at[pl.ds(s * STRIPE, STRIPE)],
                            acc.at[pl.ds(s * STRIPE, STRIPE)])
        @pl.when(c == 1)
        def _():
            pltpu.sync_copy(zeros2.at[pl.ds(s * STRIPE, STRIPE)],
                            acc.at[pl.ds(s * STRIPE, STRIPE)])

    @pl.when(s == NS - 1)
    def _():
        pltpu.sync_copy(g_hbm.at[pl.ds((NS - 1) * STRIPE, TAIL)],
                        g_tbl.at[pl.ds((NS - 1) * STRIPE, TAIL)])
        @pl.when(c == 0)
        def _():
            pltpu.sync_copy(g_hbm.at[pl.ds((NS - 1) * STRIPE, TAIL)],
                            acc.at[pl.ds((NS - 1) * STRIPE, TAIL)])
        @pl.when(c == 1)
        def _():
            pltpu.sync_copy(zeros2.at[pl.ds((NS - 1) * STRIPE, TAIL)],
                            acc.at[pl.ds((NS - 1) * STRIPE, TAIL)])

    pltpu.sync_copy(srcp.at[wid], sidx)
    pltpu.sync_copy(dstp.at[wid], didx)
    plsc.subcore_barrier()

    pltpu.async_copy(g_tbl.at[sidx.at[0]], stg_a, sem_a)

    def pair(k, carry):
        w = 2 * k
        pltpu.make_async_copy(g_tbl.at[sidx.at[w]], stg_a, sem_a).wait()
        pltpu.async_copy(g_tbl.at[sidx.at[w + 1]], stg_b, sem_b)
        pltpu.sync_copy(stg_a, acc.at[didx.at[w]], add=True)
        pltpu.make_async_copy(g_tbl.at[sidx.at[w + 1]], stg_b, sem_b).wait()

        @pl.when(k < NWIN // 2 - 1)
        def _():
            pltpu.async_copy(g_tbl.at[sidx.at[w + 2]], stg_a, sem_a)

        pltpu.sync_copy(stg_b, acc.at[didx.at[w + 1]], add=True)
        return carry

    lax.fori_loop(0, NWIN // 2, pair, 0, unroll=False)
    plsc.subcore_barrier()

    @pl.when(s < NS - 1)
    def _():
        pltpu.sync_copy(acc.at[pl.ds(s * STRIPE, STRIPE)],
                        out.at[c, pl.ds(s * STRIPE, STRIPE)])

    @pl.when(s == NS - 1)
    def _():
        pltpu.sync_copy(acc.at[pl.ds((NS - 1) * STRIPE, TAIL)],
                        out.at[c, pl.ds((NS - 1) * STRIPE, TAIL)])


def _agg_kernel(g, zeros2, srcp, dstp):
    return pl.kernel(
        _agg_body,
        out_type=jax.ShapeDtypeStruct((NC, N, H), jnp.float32),
        mesh=plsc.VectorSubcoreMesh(**_MESH),
        compiler_params=_SC_PARAMS,
        scratch_types=[
            pltpu.VMEM_SHARED((N, H), jnp.float32),
            pltpu.VMEM_SHARED((ROWS_ACC, H), jnp.float32),
            pltpu.VMEM((NWIN, WIN), jnp.int32),
            pltpu.VMEM((NWIN, WIN), jnp.int32),
            pltpu.VMEM((WIN, H), jnp.float32),
            pltpu.VMEM((WIN, H), jnp.float32),
            pltpu.SemaphoreType.DMA,
            pltpu.SemaphoreType.DMA,
        ],
    )(g, zeros2, srcp, dstp)


F = N // 2


def _tc1_body(deg4_ref, degu_ref, x_ref, w1_ref, dis_ref, g1_ref):
    d = deg4_ref[...]
    dis_e = lax.rsqrt(d[:, 0:1] + d[:, 1:2] + 1.0)
    dis_o = lax.rsqrt(d[:, 2:3] + d[:, 3:4] + 1.0)
    lane = lax.broadcasted_iota(jnp.int32, (1, 2 * H), 1)
    m_e = (lane < H).astype(jnp.float32)
    dis_ref[...] = dis_e * m_e + dis_o * (1.0 - m_e)
    du = jnp.sum(degu_ref[...], axis=1, keepdims=True)
    dis_u = lax.rsqrt(du + 1.0)
    g1_ref[...] = jnp.dot(x_ref[...], w1_ref[...],
                          preferred_element_type=jnp.float32, precision=lax.Precision.HIGHEST) * dis_u


def _tc1(deg4, degu, x, W1):
    return pl.pallas_call(
        _tc1_body,
        out_shape=(jax.ShapeDtypeStruct((F, 2 * H), jnp.float32),
                   jax.ShapeDtypeStruct((N, H), jnp.float32)),
    )(deg4, degu, x, W1)


def _tcmid_body(p_ref, dis_ref, b_ref, wblk_ref, out_ref):
    dis = dis_ref[...]
    t = p_ref[0:F, :] + p_ref[F:2 * F, :]
    h = jnp.maximum(t * dis + b_ref[...], 0.0)
    out_ref[...] = jnp.dot(h, wblk_ref[...],
                           preferred_element_type=jnp.float32, precision=lax.Precision.HIGHEST) * dis


def _tcmid(pf, dis, bf, Wblk):
    return pl.pallas_call(
        _tcmid_body,
        out_shape=jax.ShapeDtypeStruct((F, 2 * H), jnp.float32),
    )(pf, dis, bf, Wblk)


def _tc4_body(p_ref, dis_ref, b_ref, lblk_ref, lb_ref, be_ref, bo_ref,
              out_ref):
    t = p_ref[0:F, :] + p_ref[F:2 * F, :]
    h = t * dis_ref[...] + b_ref[...]
    s2 = jnp.dot(h, lblk_ref[...], preferred_element_type=jnp.float32, precision=lax.Precision.HIGHEST)
    gids = lax.broadcasted_iota(jnp.int32, (F, G), 1)
    m_e = (be_ref[...] == gids).astype(jnp.float32)
    m_o = (bo_ref[...] == gids).astype(jnp.float32)
    dn = (((0,), (0,)), ((), ()))
    seg = (lax.dot_general(m_e, s2[:, 0:1], dn,
                           preferred_element_type=jnp.float32, precision=lax.Precision.HIGHEST)
           + lax.dot_general(m_o, s2[:, 1:2], dn,
                             preferred_element_type=jnp.float32, precision=lax.Precision.HIGHEST))
    ones = jnp.ones((F, 1), jnp.float32)
    cnt = (lax.dot_general(m_e, ones, dn, preferred_element_type=jnp.float32, precision=lax.Precision.HIGHEST)
           + lax.dot_general(m_o, ones, dn,
                             preferred_element_type=jnp.float32, precision=lax.Precision.HIGHEST))
    out_ref[...] = seg / jnp.maximum(cnt, 1.0) + lb_ref[...]


def _tc4(pf, dis, bf, lin_blk, lin_b, be, bo):
    return pl.pallas_call(
        _tc4_body,
        out_shape=jax.ShapeDtypeStruct((G, 1), jnp.float32),
    )(pf, dis, bf, lin_blk, lin_b, be, bo)


def _fold_bias(b):
    return jnp.concatenate([b, b]).reshape(1, 2 * H)


def kernel(x, edge_index, batch, W1, b1, W2, b2, W3, b3, lin_W, lin_b):
    src = edge_index[0]
    dst = edge_index[1]
    srcp = jnp.concatenate(
        [src, jnp.zeros((EPAD,), jnp.int32)]).reshape(NW, NWIN, WIN)
    dstp = jnp.concatenate(
        [dst, jnp.full((EPAD,), N, jnp.int32)]).reshape(NW, NWIN, WIN)
    zeros1 = jnp.zeros((ROWS_D,), jnp.float32)
    zeros2 = jnp.zeros((N, H), jnp.float32)
    zblk = jnp.zeros((H, H), jnp.float32)
    W2blk = jnp.block([[W2, zblk], [zblk, W2]])
    W3blk = jnp.block([[W3, zblk], [zblk, W3]])
    lin_blk = jnp.block(
        [[lin_W, jnp.zeros((H, 1), jnp.float32)],
         [jnp.zeros((H, 1), jnp.float32), lin_W]])
    batch_i = batch.astype(jnp.int32)
    be = batch_i[0::2].reshape(F, 1)
    bo = batch_i[1::2].reshape(F, 1)

    degp = _deg_kernel(dstp, zeros1).reshape(NC, ROWS_D)
    degu = degp[:, :N].T
    deg4 = degu.reshape(F, 2 * NC)

    dis, g1 = _tc1(deg4, degu, x, W1)
    p1 = _agg_kernel(g1, zeros2, srcp, dstp)
    g2 = _tcmid(p1.reshape(2 * F, 2 * H), dis, _fold_bias(b1), W2blk)
    p2 = _agg_kernel(g2.reshape(N, H), zeros2, srcp, dstp)
    g3 = _tcmid(p2.reshape(2 * F, 2 * H), dis, _fold_bias(b2), W3blk)
    p3 = _agg_kernel(g3.reshape(N, H), zeros2, srcp, dstp)
    return _tc4(p3.reshape(2 * F, 2 * H), dis, _fold_bias(b3),
                lin_blk, lin_b.reshape(1, 1), be, bo)


# speedup vs baseline: 1.0722x; 1.0722x over previous
import functools

import jax
import jax.numpy as jnp
from jax import lax
from jax.experimental import pallas as pl
from jax.experimental.pallas import tpu as pltpu
from jax.experimental.pallas import tpu_sc as plsc

N = 10000
E = 320000
D_IN = 128
H = 64
G = 128

NC = 2
NS = 16
NW = NC * NS
WIN = 160
NWIN = 64
EPAD = NW * NWIN * WIN - E
ROWS_ACC = 10112
ROWS_D = 10240

ACC_PER_SUB = ROWS_ACC // NS
D_PER_SUB = ROWS_D // NS
STRIPE = 640
TAIL = N - STRIPE * (NS - 1)

_MESH = dict(core_axis_name="c", subcore_axis_name="s",
             num_cores=NC, num_subcores=NS)
_SC_PARAMS = pltpu.CompilerParams(use_tc_tiling_on_sc=False)


def _deg_body(dstp, zeros1, out, dacc, didx, ones):
    c = lax.axis_index("c")
    s = lax.axis_index("s")
    wid = s * NC + c
    pltpu.sync_copy(zeros1.at[pl.ds(s * D_PER_SUB, D_PER_SUB)],
                    dacc.at[pl.ds(s * D_PER_SUB, D_PER_SUB)])
    pltpu.sync_copy(dstp.at[wid], didx)
    for i in range(WIN // 16):
        ones[pl.ds(i * 16, 16)] = jnp.ones((16,), jnp.float32)
    plsc.subcore_barrier()

    def win(w, carry):
        pltpu.sync_copy(ones, dacc.at[didx.at[w]], add=True)
        return carry

    lax.fori_loop(0, NWIN, win, 0, unroll=False)
    plsc.subcore_barrier()
    pltpu.sync_copy(dacc.at[pl.ds(s * D_PER_SUB, D_PER_SUB)],
                    out.at[pl.ds(c * ROWS_D + s * D_PER_SUB, D_PER_SUB)])


def _deg_kernel(dstp, zeros1):
    return pl.kernel(
        _deg_body,
        out_type=jax.ShapeDtypeStruct((NC * ROWS_D,), jnp.float32),
        mesh=plsc.VectorSubcoreMesh(**_MESH),
        compiler_params=_SC_PARAMS,
        scratch_types=[
            pltpu.VMEM_SHARED((ROWS_D,), jnp.float32),
            pltpu.VMEM((NWIN, WIN), jnp.int32),
            pltpu.VMEM((WIN,), jnp.float32),
        ],
    )(dstp, zeros1)


def _agg_body(g_hbm, zeros2, srcp, dstp, out, g_tbl, acc, sidx, didx,
              stg_a, stg_b, sem_a, sem_b):
    c = lax.axis_index("c")
    s = lax.axis_index("s")
    wid = s * NC + c
    @pl.when(s < NS - 1)
    def _():
        pltpu.sync_copy(g_hbm.at[pl.ds(s * STRIPE, STRIPE)],
                        g_tbl.at[pl.ds(s * STRIPE, STRIPE)])
        @pl.when(c == 0)
        def _():
            pltpu.sync_copy(g_hbm.at[pl.ds(s * STRIPE, STRIPE)],
                            acc.at[pl.ds(s * STRIPE, STRIPE)])
        @pl.when(c == 1)
        def _():
            pltpu.sync_copy(zeros2.at[pl.ds(s * STRIPE, STRIPE)],
                            acc.at[pl.ds(s * STRIPE, STRIPE)])

    @pl.when(s == NS - 1)
    def _():
        pltpu.sync_copy(g_hbm.at[pl.ds((NS - 1) * STRIPE, TAIL)],
                        g_tbl.at[pl.ds((NS - 1) * STRIPE, TAIL)])
        @pl.when(c == 0)
        def _():
            pltpu.sync_copy(g_hbm.at[pl.ds((NS - 1) * STRIPE, TAIL)],
                            acc.at[pl.ds((NS - 1) * STRIPE, TAIL)])
        @pl.when(c == 1)
        def _():
            pltpu.sync_copy(zeros2.at[pl.ds((NS - 1) * STRIPE, TAIL)],
                            acc.at[pl.ds((NS - 1) * STRIPE, TAIL)])

    pltpu.sync_copy(srcp.at[wid], sidx)
    pltpu.sync_copy(dstp.at[wid], didx)
    plsc.subcore_barrier()

    pltpu.async_copy(g_tbl.at[sidx.at[0]], stg_a, sem_a)

    def pair(k, carry):
        w = 2 * k
        pltpu.make_async_copy(g_tbl.at[sidx.at[w]], stg_a, sem_a).wait()
        pltpu.async_copy(g_tbl.at[sidx.at[w + 1]], stg_b, sem_b)
        pltpu.sync_copy(stg_a, acc.at[didx.at[w]], add=True)
        pltpu.make_async_copy(g_tbl.at[sidx.at[w + 1]], stg_b, sem_b).wait()

        @pl.when(k < NWIN // 2 - 1)
        def _():
            pltpu.async_copy(g_tbl.at[sidx.at[w + 2]], stg_a, sem_a)

        pltpu.sync_copy(stg_b, acc.at[didx.at[w + 1]], add=True)
        return carry

    lax.fori_loop(0, NWIN // 2, pair, 0, unroll=False)
    plsc.subcore_barrier()

    @pl.when(s < NS - 1)
    def _():
        pltpu.sync_copy(acc.at[pl.ds(s * STRIPE, STRIPE)],
                        out.at[c, pl.ds(s * STRIPE, STRIPE)])

    @pl.when(s == NS - 1)
    def _():
        pltpu.sync_copy(acc.at[pl.ds((NS - 1) * STRIPE, TAIL)],
                        out.at[c, pl.ds((NS - 1) * STRIPE, TAIL)])


def _agg_kernel(g, zeros2, srcp, dstp):
    return pl.kernel(
        _agg_body,
        out_type=jax.ShapeDtypeStruct((NC, N, H), jnp.float32),
        mesh=plsc.VectorSubcoreMesh(**_MESH),
        compiler_params=_SC_PARAMS,
        scratch_types=[
            pltpu.VMEM_SHARED((N, H), jnp.float32),
            pltpu.VMEM_SHARED((ROWS_ACC, H), jnp.float32),
            pltpu.VMEM((NWIN, WIN), jnp.int32),
            pltpu.VMEM((NWIN, WIN), jnp.int32),
            pltpu.VMEM((WIN, H), jnp.float32),
            pltpu.VMEM((WIN, H), jnp.float32),
            pltpu.SemaphoreType.DMA,
            pltpu.SemaphoreType.DMA,
        ],
    )(g, zeros2, srcp, dstp)


F = N // 2


def _tc1_body(deg4_ref, degu_ref, x_ref, w1_ref, dis_ref, g1_ref):
    d = deg4_ref[...]
    dis_e = lax.rsqrt(d[:, 0:1] + d[:, 1:2] + 1.0)
    dis_o = lax.rsqrt(d[:, 2:3] + d[:, 3:4] + 1.0)
    lane = lax.broadcasted_iota(jnp.int32, (1, 2 * H), 1)
    m_e = (lane < H).astype(jnp.float32)
    dis_ref[...] = dis_e * m_e + dis_o * (1.0 - m_e)
    du = jnp.sum(degu_ref[...], axis=1, keepdims=True)
    dis_u = lax.rsqrt(du + 1.0)
    g1_ref[...] = jnp.dot(x_ref[...], w1_ref[...],
                          preferred_element_type=jnp.float32) * dis_u


def _tc1(deg4, degu, x, W1):
    return pl.pallas_call(
        _tc1_body,
        out_shape=(jax.ShapeDtypeStruct((F, 2 * H), jnp.float32),
                   jax.ShapeDtypeStruct((N, H), jnp.float32)),
    )(deg4, degu, x, W1)


def _tcmid_body(p_ref, dis_ref, b_ref, wblk_ref, out_ref):
    dis = dis_ref[...]
    t = p_ref[0:F, :] + p_ref[F:2 * F, :]
    h = jnp.maximum(t * dis + b_ref[...], 0.0)
    out_ref[...] = jnp.dot(h, wblk_ref[...],
                           preferred_element_type=jnp.float32) * dis


def _tcmid(pf, dis, bf, Wblk):
    return pl.pallas_call(
        _tcmid_body,
        out_shape=jax.ShapeDtypeStruct((F, 2 * H), jnp.float32),
    )(pf, dis, bf, Wblk)


def _tc4_body(p_ref, dis_ref, b_ref, lblk_ref, lb_ref, be_ref, bo_ref,
              out_ref):
    t = p_ref[0:F, :] + p_ref[F:2 * F, :]
    h = t * dis_ref[...] + b_ref[...]
    s2 = jnp.dot(h, lblk_ref[...], preferred_element_type=jnp.float32)
    gids = lax.broadcasted_iota(jnp.int32, (F, G), 1)
    m_e = (be_ref[...] == gids).astype(jnp.float32)
    m_o = (bo_ref[...] == gids).astype(jnp.float32)
    dn = (((0,), (0,)), ((), ()))
    seg = (lax.dot_general(m_e, s2[:, 0:1], dn,
                           preferred_element_type=jnp.float32)
           + lax.dot_general(m_o, s2[:, 1:2], dn,
                             preferred_element_type=jnp.float32))
    ones = jnp.ones((F, 1), jnp.float32)
    cnt = (lax.dot_general(m_e, ones, dn, preferred_element_type=jnp.float32)
           + lax.dot_general(m_o, ones, dn,
                             preferred_element_type=jnp.float32))
    out_ref[...] = seg / jnp.maximum(cnt, 1.0) + lb_ref[...]


def _tc4(pf, dis, bf, lin_blk, lin_b, be, bo):
    return pl.pallas_call(
        _tc4_body,
        out_shape=jax.ShapeDtypeStruct((G, 1), jnp.float32),
    )(pf, dis, bf, lin_blk, lin_b, be, bo)


def _fold_bias(b):
    return jnp.concatenate([b, b]).reshape(1, 2 * H)


def kernel(x, edge_index, batch, W1, b1, W2, b2, W3, b3, lin_W, lin_b):
    src = edge_index[0]
    dst = edge_index[1]
    srcp = jnp.concatenate(
        [src, jnp.zeros((EPAD,), jnp.int32)]).reshape(NW, NWIN, WIN)
    dstp = jnp.concatenate(
        [dst, jnp.full((EPAD,), N, jnp.int32)]).reshape(NW, NWIN, WIN)
    zeros1 = jnp.zeros((ROWS_D,), jnp.float32)
    zeros2 = jnp.zeros((N, H), jnp.float32)
    zblk = jnp.zeros((H, H), jnp.float32)
    W2blk = jnp.block([[W2, zblk], [zblk, W2]])
    W3blk = jnp.block([[W3, zblk], [zblk, W3]])
    lin_blk = jnp.block(
        [[lin_W, jnp.zeros((H, 1), jnp.float32)],
         [jnp.zeros((H, 1), jnp.float32), lin_W]])
    batch_i = batch.astype(jnp.int32)
    be = batch_i[0::2].reshape(F, 1)
    bo = batch_i[1::2].reshape(F, 1)

    degp = _deg_kernel(dstp, zeros1).reshape(NC, ROWS_D)
    degu = degp[:, :N].T
    deg4 = degu.reshape(F, 2 * NC)

    dis, g1 = _tc1(deg4, degu, x, W1)
    p1 = _agg_kernel(g1, zeros2, srcp, dstp)
    g2 = _tcmid(p1.reshape(2 * F, 2 * H), dis, _fold_bias(b1), W2blk)
    p2 = _agg_kernel(g2.reshape(N, H), zeros2, srcp, dstp)
    g3 = _tcmid(p2.reshape(2 * F, 2 * H), dis, _fold_bias(b2), W3blk)
    p3 = _agg_kernel(g3.reshape(N, H), zeros2, srcp, dstp)
    return _tc4(p3.reshape(2 * F, 2 * H), dis, _fold_bias(b3),
                lin_blk, lin_b.reshape(1, 1), be, bo)
